# Initial kernel scaffold; baseline (speedup 1.0000x reference)
import jax
import jax.numpy as jnp
from jax.experimental import pallas as pl


def kernel(x, states, edge_index, edge_weight, node_bias, node_response):
    raise NotImplementedError("write your pallas kernel here")


# speedup vs baseline: 4.4365x; 4.4365x over previous
import functools

import jax
import jax.numpy as jnp
from jax import lax
from jax.experimental import pallas as pl
from jax.experimental.pallas import tpu as pltpu
from jax.experimental.pallas import tpu_sc as plsc

NC = 2
NS = 16
NW = NC * NS
CHUNK = 80
OUT_SIZE = 64


def _sc_edge_scatter(st_t, src, dst, w, zeros, n_nodes):
    n_edges = src.shape[0]
    b = st_t.shape[1]
    epw = n_edges // NW
    n_chunks = epw // CHUNK
    zrows = zeros.shape[0]
    nslices = n_nodes // zrows
    mesh = plsc.VectorSubcoreMesh(core_axis_name="c", subcore_axis_name="s")

    @functools.partial(
        pl.kernel,
        out_type=jax.ShapeDtypeStruct((NC, n_nodes, b), jnp.float32),
        mesh=mesh,
        scratch_types=[
            pltpu.VMEM((CHUNK,), jnp.int32),
            pltpu.VMEM((CHUNK,), jnp.int32),
            pltpu.VMEM((CHUNK,), jnp.float32),
            pltpu.VMEM((CHUNK, b), jnp.float32),
            pltpu.VMEM_SHARED((n_nodes, b), jnp.float32),
            pltpu.SemaphoreType.DMA,
        ],
        compiler_params=pltpu.CompilerParams(use_tc_tiling_on_sc=False),
    )
    def sc_kern(st_hbm, src_hbm, dst_hbm, w_hbm, zeros_hbm, out_hbm,
                src_v, dst_v, w_v, rows_v, acc_sh, sem):
        cid = lax.axis_index("c")
        sid = lax.axis_index("s")
        wid = sid * NC + cid

        @pl.when(sid < nslices)
        def _zero():
            pltpu.sync_copy(zeros_hbm,
                            acc_sh.at[pl.ds(sid * zrows, zrows)])
        plsc.subcore_barrier()

        def chunk_body(c, _):
            e0 = wid * epw + c * CHUNK
            pltpu.sync_copy(src_hbm.at[pl.ds(e0, CHUNK)], src_v)
            pltpu.sync_copy(dst_hbm.at[pl.ds(e0, CHUNK)], dst_v)
            pltpu.sync_copy(w_hbm.at[pl.ds(e0, CHUNK)], w_v)
            pltpu.async_copy(st_hbm.at[src_v], rows_v, sem).wait()

            def group_body(g, _):
                wvec = w_v[pl.ds(g * 16, 16)]
                for l in range(16):
                    wb = jnp.broadcast_to(wvec[l], (16,))
                    i = g * 16 + l
                    for j in range(b // 16):
                        sl = pl.ds(j * 16, 16)
                        rows_v[i, sl] = rows_v[i, sl] * wb
                return 0
            lax.fori_loop(0, CHUNK // 16, group_body, 0)

            pltpu.sync_copy(rows_v, acc_sh.at[dst_v], add=True)
            return 0
        lax.fori_loop(0, n_chunks, chunk_body, 0)

        plsc.subcore_barrier()

        @pl.when(sid < nslices)
        def _writeback():
            pltpu.sync_copy(acc_sh.at[pl.ds(sid * zrows, zrows)],
                            out_hbm.at[cid, pl.ds(sid * zrows, zrows)])

    return sc_kern(st_t, src, dst, w, zeros)


def _tc_finish(partials, bias2, resp2):
    n, b = partials.shape[1], partials.shape[2]

    def tc_kern(p_ref, b_ref, r_ref, o_ref):
        agg = p_ref[0] + p_ref[1]
        o_ref[...] = jnp.tanh(b_ref[...] + r_ref[...] * agg)

    return pl.pallas_call(
        tc_kern,
        out_shape=jax.ShapeDtypeStruct((n, b), jnp.float32),
    )(partials, bias2, resp2)


@jax.jit
def kernel(x, states, edge_index, edge_weight, node_bias, node_response):
    n_nodes = node_bias.shape[0]
    in_size = x.shape[1]
    st = states.at[:, :in_size].set(x)
    st_t = st.T
    zeros = jnp.zeros((1000, st_t.shape[1]), jnp.float32)
    partials = _sc_edge_scatter(st_t, edge_index[0], edge_index[1],
                                edge_weight, zeros, n_nodes)
    act = _tc_finish(partials,
                     node_bias.reshape(n_nodes, 1),
                     node_response.reshape(n_nodes, 1))
    new_states = act.T
    new_states = new_states.at[:, :in_size].set(x)
    y = new_states[:, -OUT_SIZE:]
    return (y, new_states)


# speedup vs baseline: 6.7328x; 1.5176x over previous
import functools

import jax
import jax.numpy as jnp
from jax import lax
from jax.experimental import pallas as pl
from jax.experimental.pallas import tpu as pltpu
from jax.experimental.pallas import tpu_sc as plsc

NC = 2
NS = 16
NW = NC * NS
CHUNK = 80
OUT_SIZE = 64


def _sc_edge_scatter(st_t, src3, dst3, w3, zeros, n_nodes):
    b = st_t.shape[1]
    n_chunks = src3.shape[1]
    zrows = zeros.shape[0]
    nslices = n_nodes // zrows
    mesh = plsc.VectorSubcoreMesh(core_axis_name="c", subcore_axis_name="s")

    @functools.partial(
        pl.kernel,
        out_type=jax.ShapeDtypeStruct((NC, n_nodes, b), jnp.float32),
        mesh=mesh,
        scratch_types=[
            pltpu.VMEM((n_chunks, CHUNK), jnp.int32),
            pltpu.VMEM((n_chunks, CHUNK), jnp.int32),
            pltpu.VMEM((n_chunks, CHUNK), jnp.float32),
            pltpu.VMEM((CHUNK, b), jnp.float32),
            pltpu.VMEM((CHUNK, b), jnp.float32),
            pltpu.VMEM_SHARED((n_nodes, b), jnp.float32),
            pltpu.SemaphoreType.DMA,
            pltpu.SemaphoreType.DMA,
        ],
        compiler_params=pltpu.CompilerParams(use_tc_tiling_on_sc=False),
    )
    def sc_kern(st_hbm, src_hbm, dst_hbm, w_hbm, zeros_hbm, out_hbm,
                src_v, dst_v, w_v, rows_a, rows_b, acc_sh, sem_a, sem_b):
        cid = lax.axis_index("c")
        sid = lax.axis_index("s")
        wid = sid * NC + cid

        pltpu.sync_copy(src_hbm.at[wid], src_v)
        pltpu.sync_copy(dst_hbm.at[wid], dst_v)
        pltpu.sync_copy(w_hbm.at[wid], w_v)

        @pl.when(sid < nslices)
        def _zero():
            pltpu.sync_copy(zeros_hbm,
                            acc_sh.at[pl.ds(sid * zrows, zrows)])
        plsc.subcore_barrier()

        def gather_start(c, buf, sem):
            pltpu.async_copy(st_hbm.at[src_v.at[c]], buf, sem)

        def gather_wait(buf, sem):
            pltpu.make_async_copy(st_hbm.at[src_v.at[0]], buf, sem).wait()

        def scale_scatter(c, buf):
            def group_body(g, _):
                wvec = w_v[c, pl.ds(g * 16, 16)]
                for l in range(16):
                    wb = jnp.broadcast_to(wvec[l], (16,))
                    i = g * 16 + l
                    for j in range(b // 16):
                        sl = pl.ds(j * 16, 16)
                        buf[i, sl] = buf[i, sl] * wb
                return 0
            lax.fori_loop(0, CHUNK // 16, group_body, 0)
            pltpu.sync_copy(buf, acc_sh.at[dst_v.at[c]], add=True)

        gather_start(0, rows_a, sem_a)

        def pair_body(k, _):
            c0 = 2 * k
            gather_wait(rows_a, sem_a)
            gather_start(c0 + 1, rows_b, sem_b)
            scale_scatter(c0, rows_a)
            gather_wait(rows_b, sem_b)
            gather_start(c0 + 2, rows_a, sem_a)
            scale_scatter(c0 + 1, rows_b)
            return 0
        lax.fori_loop(0, (n_chunks - 1) // 2, pair_body, 0)

        gather_wait(rows_a, sem_a)
        scale_scatter(n_chunks - 1, rows_a)

        plsc.subcore_barrier()

        @pl.when(sid < nslices)
        def _writeback():
            pltpu.sync_copy(acc_sh.at[pl.ds(sid * zrows, zrows)],
                            out_hbm.at[cid, pl.ds(sid * zrows, zrows)])

    return sc_kern(st_t, src3, dst3, w3, zeros)


def _tc_finish(partials, bias2, resp2):
    n, b = partials.shape[1], partials.shape[2]

    def tc_kern(p_ref, b_ref, r_ref, o_ref):
        agg = p_ref[0] + p_ref[1]
        o_ref[...] = jnp.tanh(b_ref[...] + r_ref[...] * agg)

    return pl.pallas_call(
        tc_kern,
        out_shape=jax.ShapeDtypeStruct((n, b), jnp.float32),
    )(partials, bias2, resp2)


@jax.jit
def kernel(x, states, edge_index, edge_weight, node_bias, node_response):
    n_nodes = node_bias.shape[0]
    n_edges = edge_weight.shape[0]
    in_size = x.shape[1]
    st = states.at[:, :in_size].set(x)
    st_t = st.T
    n_chunks = n_edges // (NW * CHUNK)
    src3 = edge_index[0].reshape(NW, n_chunks, CHUNK)
    dst3 = edge_index[1].reshape(NW, n_chunks, CHUNK)
    w3 = edge_weight.reshape(NW, n_chunks, CHUNK)
    zeros = jnp.zeros((1000, st_t.shape[1]), jnp.float32)
    partials = _sc_edge_scatter(st_t, src3, dst3, w3, zeros, n_nodes)
    act = _tc_finish(partials,
                     node_bias.reshape(n_nodes, 1),
                     node_response.reshape(n_nodes, 1))
    new_states = act.T
    new_states = new_states.at[:, :in_size].set(x)
    y = new_states[:, -OUT_SIZE:]
    return (y, new_states)
